# baseline (speedup 1.0000x reference)
import jax
import jax.numpy as jnp
from jax import lax
from jax.experimental import pallas as pl
from jax.experimental.pallas import tpu as pltpu

_NUM_PILOT = 512
_CHUNK = 128


def _find_thresh(w8):
    u = lax.bitcast_convert_type(w8, jnp.uint32)
    top = jnp.uint32(0x80000000)
    key = jnp.where(u >= top, ~u, u | top)

    jj = jnp.arange(16, dtype=jnp.uint32).reshape(16, 1, 1)
    p = jnp.zeros((1, 1), jnp.uint32)
    bit = jnp.full((1, 1), jnp.uint32(1) << 28, jnp.uint32)
    for _ in range(8):
        cands = p.reshape(1, 1, 1) + jj * bit.reshape(1, 1, 1)
        cnt = jnp.sum((key[None] >= cands).astype(jnp.int32), axis=(1, 2))
        m = jnp.sum((cnt >= _NUM_PILOT).astype(jnp.int32), keepdims=True)
        nib = (m.reshape(1, 1) - 1).astype(jnp.uint32)
        p = p + nib * bit
        bit = lax.shift_right_logical(bit, jnp.uint32(4))
    t = jnp.where(p >= top, p ^ top, ~p)
    return lax.bitcast_convert_type(t, jnp.float32)


def _body(x_hbm, w_ref, w8_ref, o_hbm, ibuf, obuf, isem, osem):
    b = x_hbm.shape[0]
    n = b // _CHUNK

    def in_copy(k):
        return pltpu.make_async_copy(
            x_hbm.at[pl.ds(k * _CHUNK, _CHUNK)], ibuf.at[k % 2], isem.at[k % 2]
        )

    def out_copy(k):
        return pltpu.make_async_copy(
            obuf.at[k % 2], o_hbm.at[pl.ds(k * _CHUNK, _CHUNK)], osem.at[k % 2]
        )

    in_copy(0).start()
    if n > 1:
        in_copy(1).start()

    thresh = _find_thresh(w8_ref[...])
    mask = (w_ref[...] > thresh).astype(jnp.float32)

    for k in range(n):
        in_copy(k).wait()
        if k >= 2:
            out_copy(k - 2).wait()
        obuf[k % 2] = ibuf[k % 2] * mask
        out_copy(k).start()
        if k + 2 < n:
            in_copy(k + 2).start()
    for k in range(max(n - 2, 0), n):
        out_copy(k).wait()


def kernel(inputs, kernel):
    b, d = inputs.shape
    w8 = kernel.reshape(8, d // 8)
    out = pl.pallas_call(
        _body,
        in_specs=[
            pl.BlockSpec(memory_space=pl.ANY),
            pl.BlockSpec(memory_space=pltpu.VMEM),
            pl.BlockSpec(memory_space=pltpu.VMEM),
        ],
        out_specs=pl.BlockSpec(memory_space=pl.ANY),
        scratch_shapes=[
            pltpu.VMEM((2, _CHUNK, d), jnp.float32),
            pltpu.VMEM((2, _CHUNK, d), jnp.float32),
            pltpu.SemaphoreType.DMA((2,)),
            pltpu.SemaphoreType.DMA((2,)),
        ],
        out_shape=jax.ShapeDtypeStruct(inputs.shape, inputs.dtype),
    )(inputs, kernel, w8)
    return out


# speedup vs baseline: 1.0817x; 1.0817x over previous
import jax
import jax.numpy as jnp
from jax import lax
from jax.experimental import pallas as pl
from jax.experimental.pallas import tpu as pltpu

_NUM_PILOT = 512
_CHUNK = 64


def _find_thresh(w8):
    u = lax.bitcast_convert_type(w8, jnp.uint32)
    top = jnp.uint32(0x80000000)
    key = jnp.where(u >= top, ~u, u | top)

    jj = jnp.arange(16, dtype=jnp.uint32).reshape(16, 1, 1)
    p = jnp.zeros((1, 1), jnp.uint32)
    bit = jnp.full((1, 1), jnp.uint32(1) << 28, jnp.uint32)
    for _ in range(8):
        cands = p.reshape(1, 1, 1) + jj * bit.reshape(1, 1, 1)
        cnt = jnp.sum((key[None] >= cands).astype(jnp.int32), axis=(1, 2))
        m = jnp.sum((cnt >= _NUM_PILOT).astype(jnp.int32), keepdims=True)
        nib = (m.reshape(1, 1) - 1).astype(jnp.uint32)
        p = p + nib * bit
        bit = lax.shift_right_logical(bit, jnp.uint32(4))
    t = jnp.where(p >= top, p ^ top, ~p)
    return lax.bitcast_convert_type(t, jnp.float32)


def _body(x_hbm, w_ref, w8_ref, o_hbm, ibuf, obuf, isem, osem):
    b = x_hbm.shape[0]
    n = b // _CHUNK

    def in_copy(k):
        return pltpu.make_async_copy(
            x_hbm.at[pl.ds(k * _CHUNK, _CHUNK)], ibuf.at[k % 2], isem.at[k % 2]
        )

    def out_copy(k):
        return pltpu.make_async_copy(
            obuf.at[k % 2], o_hbm.at[pl.ds(k * _CHUNK, _CHUNK)], osem.at[k % 2]
        )

    in_copy(0).start()
    if n > 1:
        in_copy(1).start()

    thresh = _find_thresh(w8_ref[...])
    mask = (w_ref[...] > thresh).astype(jnp.float32)

    for k in range(n):
        in_copy(k).wait()
        if k >= 2:
            out_copy(k - 2).wait()
        obuf[k % 2] = ibuf[k % 2] * mask
        out_copy(k).start()
        if k + 2 < n:
            in_copy(k + 2).start()
    for k in range(max(n - 2, 0), n):
        out_copy(k).wait()


def kernel(inputs, kernel):
    b, d = inputs.shape
    w8 = kernel.reshape(8, d // 8)
    out = pl.pallas_call(
        _body,
        in_specs=[
            pl.BlockSpec(memory_space=pl.ANY),
            pl.BlockSpec(memory_space=pltpu.VMEM),
            pl.BlockSpec(memory_space=pltpu.VMEM),
        ],
        out_specs=pl.BlockSpec(memory_space=pl.ANY),
        scratch_shapes=[
            pltpu.VMEM((2, _CHUNK, d), jnp.float32),
            pltpu.VMEM((2, _CHUNK, d), jnp.float32),
            pltpu.SemaphoreType.DMA((2,)),
            pltpu.SemaphoreType.DMA((2,)),
        ],
        out_shape=jax.ShapeDtypeStruct(inputs.shape, inputs.dtype),
    )(inputs, kernel, w8)
    return out
